# Initial kernel scaffold; baseline (speedup 1.0000x reference)
import functools

import jax
import jax.numpy as jnp
from jax import lax
from jax.experimental import pallas as pl
from jax.experimental.pallas import tpu as pltpu
from jax.experimental.pallas import tpu_sc as plsc

N = 10000
D = 256
HALF = 128
E = 160000
CHUNK = 128
E_PAD = 163840
ROWS = 10240
PAD_DST = 10008
NS = 16
NC = 2

_mesh = plsc.VectorSubcoreMesh(core_axis_name="c", subcore_axis_name="s")


def _zero_block(zblk):
    w = zblk.shape[1]
    zv = jnp.zeros((16,), jnp.float32)

    def row(i, _):
        for j in range(w // 16):
            zblk[i, pl.ds(j * 16, 16)] = zv
        return 0

    lax.fori_loop(0, zblk.shape[0], row, 0)


@functools.partial(
    pl.kernel,
    out_type=jax.ShapeDtypeStruct((NC * ROWS, 16), jnp.float32),
    mesh=_mesh,
    scratch_types=[
        pltpu.VMEM((CHUNK, 16), jnp.float32),
        pltpu.VMEM((40, CHUNK), jnp.int32),
        pltpu.VMEM_SHARED((ROWS, 16), jnp.float32),
    ],
)
def _deg(dst_hbm, out_hbm, ones_v, didx_v, acc):
    c = lax.axis_index("c")
    s = lax.axis_index("s")
    wid = c * NS + s
    _zero_block(ones_v)
    for z in range(5):
        pltpu.sync_copy(ones_v, acc.at[pl.ds(s * 640 + z * CHUNK, CHUNK)])
    ov = jnp.full((16,), 1.0, jnp.float32)
    for i in range(CHUNK):
        ones_v[i, :] = ov
    plsc.subcore_barrier()

    pltpu.sync_copy(dst_hbm.at[pl.ds(wid * 40, 40)], didx_v)

    def body(j, _):
        pltpu.sync_copy(ones_v, acc.at[didx_v.at[j]], add=True)
        return 0

    lax.fori_loop(0, 40, body, 0)
    plsc.subcore_barrier()
    pltpu.sync_copy(
        acc.at[pl.ds(s * 640, 640)],
        out_hbm.at[pl.ds(c * ROWS + s * 640, 640)],
    )


@functools.partial(
    pl.kernel,
    out_type=jax.ShapeDtypeStruct((NC * ROWS, HALF), jnp.float32),
    mesh=_mesh,
    scratch_types=[
        pltpu.VMEM((80, CHUNK), jnp.int32),
        pltpu.VMEM((80, CHUNK), jnp.int32),
        pltpu.VMEM((CHUNK, HALF), jnp.float32),
        pltpu.VMEM((CHUNK, HALF), jnp.float32),
        pltpu.VMEM_SHARED((ROWS, HALF), jnp.float32),
        pltpu.SemaphoreType.DMA,
    ],
)
def _agg(src_hbm, dst_hbm, hs_hbm, out_hbm, sidx_v, didx_v, rows_v, zblk, acc, sem):
    c = lax.axis_index("c")
    s = lax.axis_index("s")
    _zero_block(zblk)
    for z in range(5):
        pltpu.sync_copy(zblk, acc.at[pl.ds(s * 640 + z * CHUNK, CHUNK)])
    plsc.subcore_barrier()

    pltpu.sync_copy(src_hbm.at[pl.ds(c * 1280 + s * 80, 80)], sidx_v)
    pltpu.sync_copy(dst_hbm.at[pl.ds(s * 80, 80)], didx_v)

    def body(j, _):
        pltpu.async_copy(hs_hbm.at[sidx_v.at[j]], rows_v, sem).wait()
        pltpu.sync_copy(rows_v, acc.at[didx_v.at[j]], add=True)
        return 0

    lax.fori_loop(0, 80, body, 0)
    plsc.subcore_barrier()
    pltpu.sync_copy(
        acc.at[pl.ds(s * 640, 640)],
        out_hbm.at[pl.ds(c * ROWS + s * 640, 640)],
    )


def _mm_body(x_ref, w_ref, degp_ref, hs_ref, self_ref):
    h = jnp.dot(x_ref[...], w_ref[...], preferred_element_type=jnp.float32)
    deg = degp_ref[0, :, 0:1] + degp_ref[1, :, 0:1] + 1.0
    dis = lax.rsqrt(deg)
    hs_ref[0] = h[:, :HALF] * dis
    hs_ref[1] = h[:, HALF:] * dis
    self_ref[...] = h / deg


def _mm(x, W, degp):
    blk = 1000
    return pl.pallas_call(
        _mm_body,
        grid=(N // blk,),
        in_specs=[
            pl.BlockSpec((blk, D), lambda i: (i, 0)),
            pl.BlockSpec((D, D), lambda i: (0, 0)),
            pl.BlockSpec((2, blk, 16), lambda i: (0, i, 0)),
        ],
        out_specs=[
            pl.BlockSpec((2, blk, HALF), lambda i: (0, i, 0)),
            pl.BlockSpec((blk, D), lambda i: (i, 0)),
        ],
        out_shape=[
            jax.ShapeDtypeStruct((2, N, HALF), jnp.float32),
            jax.ShapeDtypeStruct((N, D), jnp.float32),
        ],
    )(x, W, degp)


def _bn_body(agg_ref, degp_ref, self_ref, b_ref, g_ref, be_ref, o_ref):
    deg = degp_ref[0, :N, 0:1] + degp_ref[1, :N, 0:1] + 1.0
    dis = lax.rsqrt(deg)
    out = agg_ref[0, :N, :] * dis + self_ref[...] + b_ref[...]
    m = jnp.mean(out, axis=0, keepdims=True)
    v = jnp.mean((out - m) * (out - m), axis=0, keepdims=True)
    xn = (out - m) / jnp.sqrt(v + 1e-5)
    y = g_ref[...] * xn + be_ref[...]
    o_ref[...] = jnp.maximum(y, 0.0)


def _bn(agg, degp, self_t, b2, g2, be2):
    return pl.pallas_call(
        _bn_body,
        grid=(2,),
        in_specs=[
            pl.BlockSpec((1, ROWS, HALF), lambda i: (i, 0, 0)),
            pl.BlockSpec((2, ROWS, 16), lambda i: (0, 0, 0)),
            pl.BlockSpec((N, HALF), lambda i: (0, i)),
            pl.BlockSpec((1, HALF), lambda i: (0, i)),
            pl.BlockSpec((1, HALF), lambda i: (0, i)),
            pl.BlockSpec((1, HALF), lambda i: (0, i)),
        ],
        out_specs=pl.BlockSpec((N, HALF), lambda i: (0, i)),
        out_shape=jax.ShapeDtypeStruct((N, D), jnp.float32),
    )(agg, degp, self_t, b2, g2, be2)


def kernel(x, edge_index, W, b, gamma, beta):
    src = edge_index[0].astype(jnp.int32)
    dst = edge_index[1].astype(jnp.int32)
    pad = E_PAD - E
    src_p = jnp.concatenate([src, jnp.zeros((pad,), jnp.int32)])
    dst_p = jnp.concatenate([dst, jnp.full((pad,), PAD_DST, jnp.int32)])
    src2 = jnp.stack([src_p, src_p + N]).reshape(2 * 1280, CHUNK)
    dst2 = dst_p.reshape(1280, CHUNK)

    degp = _deg(dst2).reshape(2, ROWS, 16)
    hs, self_t = _mm(x, W, degp)
    agg = _agg(src2, dst2, hs.reshape(2 * N, HALF)).reshape(2, ROWS, HALF)
    return _bn(
        agg, degp, self_t,
        b.reshape(1, D), gamma.reshape(1, D), beta.reshape(1, D),
    )


# speedup vs baseline: 9.5895x; 9.5895x over previous
import functools

import jax
import jax.numpy as jnp
from jax import lax
from jax.experimental import pallas as pl
from jax.experimental.pallas import tpu as pltpu
from jax.experimental.pallas import tpu_sc as plsc

N = 10000
D = 256
HALF = 128
E = 160000
CHUNK = 128
E_PAD = 163840
ROWS = 10240
PAD_DST = 10008
NS = 16
NC = 2

_mesh = plsc.VectorSubcoreMesh(core_axis_name="c", subcore_axis_name="s")


def _zero_block(zblk):
    w = zblk.shape[1]
    zv = jnp.zeros((16,), jnp.float32)

    def row(i, _):
        for j in range(w // 16):
            zblk[i, pl.ds(j * 16, 16)] = zv
        return 0

    lax.fori_loop(0, zblk.shape[0], row, 0)


@functools.partial(
    pl.kernel,
    out_type=jax.ShapeDtypeStruct((NC * ROWS, 16), jnp.float32),
    mesh=_mesh,
    scratch_types=[
        pltpu.VMEM((CHUNK, 16), jnp.float32),
        pltpu.VMEM((40, CHUNK), jnp.int32),
        pltpu.VMEM_SHARED((ROWS, 16), jnp.float32),
    ],
)
def _deg(dst_hbm, out_hbm, ones_v, didx_v, acc):
    c = lax.axis_index("c")
    s = lax.axis_index("s")
    wid = c * NS + s
    _zero_block(ones_v)
    for z in range(5):
        pltpu.sync_copy(ones_v, acc.at[pl.ds(s * 640 + z * CHUNK, CHUNK)])
    ov = jnp.full((16,), 1.0, jnp.float32)
    for i in range(CHUNK):
        ones_v[i, :] = ov
    plsc.subcore_barrier()

    pltpu.sync_copy(dst_hbm.at[pl.ds(wid * 40, 40)], didx_v)

    def body(j, _):
        pltpu.sync_copy(ones_v, acc.at[didx_v.at[j]], add=True)
        return 0

    lax.fori_loop(0, 40, body, 0)
    plsc.subcore_barrier()
    pltpu.sync_copy(
        acc.at[pl.ds(s * 640, 640)],
        out_hbm.at[pl.ds(c * ROWS + s * 640, 640)],
    )


@functools.partial(
    pl.kernel,
    out_type=jax.ShapeDtypeStruct((NC * ROWS, HALF), jnp.float32),
    mesh=_mesh,
    scratch_types=[
        pltpu.VMEM((80, CHUNK), jnp.int32),
        pltpu.VMEM((80, CHUNK), jnp.int32),
        pltpu.VMEM((CHUNK, HALF), jnp.float32),
        pltpu.VMEM_SHARED((ROWS, HALF), jnp.float32),
        pltpu.SemaphoreType.DMA,
    ],
)
def _agg(src_hbm, dst_hbm, hs_hbm, out_hbm, sidx_v, didx_v, rows_v, acc, sem):
    c = lax.axis_index("c")
    s = lax.axis_index("s")
    _zero_block(rows_v)
    for z in range(5):
        pltpu.sync_copy(rows_v, acc.at[pl.ds(s * 640 + z * CHUNK, CHUNK)])
    plsc.subcore_barrier()

    pltpu.sync_copy(src_hbm.at[pl.ds(c * 1280 + s * 80, 80)], sidx_v)
    pltpu.sync_copy(dst_hbm.at[pl.ds(s * 80, 80)], didx_v)

    def body(j, _):
        pltpu.async_copy(hs_hbm.at[sidx_v.at[j]], rows_v, sem).wait()
        pltpu.sync_copy(rows_v, acc.at[didx_v.at[j]], add=True)
        return 0

    lax.fori_loop(0, 80, body, 0)
    plsc.subcore_barrier()
    pltpu.sync_copy(
        acc.at[pl.ds(s * 640, 640)],
        out_hbm.at[pl.ds(c * ROWS + s * 640, 640)],
    )


def _mm_body(x_ref, w_ref, degp_ref, hs_ref, self_ref):
    h = jnp.dot(x_ref[...], w_ref[...], preferred_element_type=jnp.float32)
    deg = degp_ref[0, :, 0:1] + degp_ref[1, :, 0:1] + 1.0
    dis = lax.rsqrt(deg)
    hs_ref[0] = h[:, :HALF] * dis
    hs_ref[1] = h[:, HALF:] * dis
    self_ref[...] = h / deg


def _mm(x, W, degp):
    blk = 1000
    return pl.pallas_call(
        _mm_body,
        grid=(N // blk,),
        in_specs=[
            pl.BlockSpec((blk, D), lambda i: (i, 0)),
            pl.BlockSpec((D, D), lambda i: (0, 0)),
            pl.BlockSpec((2, blk, 16), lambda i: (0, i, 0)),
        ],
        out_specs=[
            pl.BlockSpec((2, blk, HALF), lambda i: (0, i, 0)),
            pl.BlockSpec((blk, D), lambda i: (i, 0)),
        ],
        out_shape=[
            jax.ShapeDtypeStruct((2, N, HALF), jnp.float32),
            jax.ShapeDtypeStruct((N, D), jnp.float32),
        ],
    )(x, W, degp)


def _bn_body(agg_ref, degp_ref, self_ref, b_ref, g_ref, be_ref, o_ref):
    deg = degp_ref[0, :N, 0:1] + degp_ref[1, :N, 0:1] + 1.0
    dis = lax.rsqrt(deg)
    out = agg_ref[0, :N, :] * dis + self_ref[...] + b_ref[...]
    m = jnp.mean(out, axis=0, keepdims=True)
    v = jnp.mean((out - m) * (out - m), axis=0, keepdims=True)
    xn = (out - m) / jnp.sqrt(v + 1e-5)
    y = g_ref[...] * xn + be_ref[...]
    o_ref[...] = jnp.maximum(y, 0.0)


def _bn(agg, degp, self_t, b2, g2, be2):
    return pl.pallas_call(
        _bn_body,
        grid=(2,),
        in_specs=[
            pl.BlockSpec((1, ROWS, HALF), lambda i: (i, 0, 0)),
            pl.BlockSpec((2, ROWS, 16), lambda i: (0, 0, 0)),
            pl.BlockSpec((N, HALF), lambda i: (0, i)),
            pl.BlockSpec((1, HALF), lambda i: (0, i)),
            pl.BlockSpec((1, HALF), lambda i: (0, i)),
            pl.BlockSpec((1, HALF), lambda i: (0, i)),
        ],
        out_specs=pl.BlockSpec((N, HALF), lambda i: (0, i)),
        out_shape=jax.ShapeDtypeStruct((N, D), jnp.float32),
    )(agg, degp, self_t, b2, g2, be2)


def kernel(x, edge_index, W, b, gamma, beta):
    src = edge_index[0].astype(jnp.int32)
    dst = edge_index[1].astype(jnp.int32)
    pad = E_PAD - E
    src_p = jnp.concatenate([src, jnp.zeros((pad,), jnp.int32)])
    dst_p = jnp.concatenate([dst, jnp.full((pad,), PAD_DST, jnp.int32)])
    src2 = jnp.stack([src_p, src_p + N]).reshape(2 * 1280, CHUNK)
    dst2 = dst_p.reshape(1280, CHUNK)

    degp = _deg(dst2).reshape(2, ROWS, 16)
    hs, self_t = _mm(x, W, degp)
    agg = _agg(src2, dst2, hs.reshape(2 * N, HALF)).reshape(2, ROWS, HALF)
    return _bn(
        agg, degp, self_t,
        b.reshape(1, D), gamma.reshape(1, D), beta.reshape(1, D),
    )


# speedup vs baseline: 11.2067x; 1.1686x over previous
import functools

import jax
import jax.numpy as jnp
from jax import lax
from jax.experimental import pallas as pl
from jax.experimental.pallas import tpu as pltpu
from jax.experimental.pallas import tpu_sc as plsc

N = 10000
D = 256
HALF = 128
E = 160000
CHUNK = 128
E_PAD = 163840
ROWS = 10240
PAD_DST = 10008
NS = 16
NC = 2

_mesh = plsc.VectorSubcoreMesh(core_axis_name="c", subcore_axis_name="s")


def _zero_block(zblk):
    w = zblk.shape[1]
    zv = jnp.zeros((16,), jnp.float32)

    def row(i, _):
        for j in range(w // 16):
            zblk[i, pl.ds(j * 16, 16)] = zv
        return 0

    lax.fori_loop(0, zblk.shape[0], row, 0)


@functools.partial(
    pl.kernel,
    out_type=jax.ShapeDtypeStruct((NC * ROWS, 16), jnp.float32),
    mesh=_mesh,
    scratch_types=[
        pltpu.VMEM((CHUNK, 16), jnp.float32),
        pltpu.VMEM((40, CHUNK), jnp.int32),
        pltpu.VMEM_SHARED((ROWS, 16), jnp.float32),
    ],
)
def _deg(dst_hbm, out_hbm, ones_v, didx_v, acc):
    c = lax.axis_index("c")
    s = lax.axis_index("s")
    wid = c * NS + s
    _zero_block(ones_v)
    for z in range(5):
        pltpu.sync_copy(ones_v, acc.at[pl.ds(s * 640 + z * CHUNK, CHUNK)])
    ov = jnp.full((16,), 1.0, jnp.float32)
    for i in range(CHUNK):
        ones_v[i, :] = ov
    plsc.subcore_barrier()

    pltpu.sync_copy(dst_hbm.at[pl.ds(wid * 40, 40)], didx_v)

    def body(j, _):
        pltpu.sync_copy(ones_v, acc.at[didx_v.at[j]], add=True)
        return 0

    lax.fori_loop(0, 40, body, 0)
    plsc.subcore_barrier()
    pltpu.sync_copy(
        acc.at[pl.ds(s * 640, 640)],
        out_hbm.at[pl.ds(c * ROWS + s * 640, 640)],
    )


@functools.partial(
    pl.kernel,
    out_type=jax.ShapeDtypeStruct((NC * ROWS, HALF), jnp.float32),
    mesh=_mesh,
    scratch_types=[
        pltpu.VMEM((40, CHUNK), jnp.int32),
        pltpu.VMEM((40, CHUNK), jnp.int32),
        pltpu.VMEM((CHUNK, HALF), jnp.float32),
        pltpu.VMEM((CHUNK, HALF), jnp.float32),
        pltpu.VMEM_SHARED((ROWS, HALF), jnp.float32),
        pltpu.SemaphoreType.DMA,
        pltpu.SemaphoreType.DMA,
    ],
)
def _agg(src_hbm, dst_hbm, hs_hbm, out_hbm, sidx_v, didx_v, r0, r1, acc, sa, sb):
    c = lax.axis_index("c")
    s = lax.axis_index("s")
    _zero_block(r0)
    for z in range(5):
        pltpu.sync_copy(r0, acc.at[pl.ds(s * 640 + z * CHUNK, CHUNK)])
    plsc.subcore_barrier()

    for h in range(2):
        pltpu.sync_copy(src_hbm.at[pl.ds(c * 1280 + s * 80 + h * 40, 40)], sidx_v)
        pltpu.sync_copy(dst_hbm.at[pl.ds(s * 80 + h * 40, 40)], didx_v)
        pltpu.async_copy(hs_hbm.at[sidx_v.at[0]], r0, sa)

        def body(jj, _):
            e = 2 * jj
            ga = pltpu.make_async_copy(hs_hbm.at[sidx_v.at[e]], r0, sa)
            gb = pltpu.async_copy(hs_hbm.at[sidx_v.at[e + 1]], r1, sb)
            ga.wait()
            pltpu.sync_copy(r0, acc.at[didx_v.at[e]], add=True)

            @pl.when(jj < 19)
            def _():
                pltpu.async_copy(hs_hbm.at[sidx_v.at[e + 2]], r0, sa)

            gb.wait()
            pltpu.sync_copy(r1, acc.at[didx_v.at[e + 1]], add=True)
            return 0

        lax.fori_loop(0, 20, body, 0)
    plsc.subcore_barrier()
    pltpu.sync_copy(
        acc.at[pl.ds(s * 640, 640)],
        out_hbm.at[pl.ds(c * ROWS + s * 640, 640)],
    )


def _mm_body(x_ref, w_ref, degp_ref, hs_ref, self_ref):
    h = jnp.dot(x_ref[...], w_ref[...], preferred_element_type=jnp.float32)
    deg = degp_ref[0, :, 0:1] + degp_ref[1, :, 0:1] + 1.0
    dis = lax.rsqrt(deg)
    hs_ref[0] = h[:, :HALF] * dis
    hs_ref[1] = h[:, HALF:] * dis
    self_ref[...] = h / deg


def _mm(x, W, degp):
    blk = 1000
    return pl.pallas_call(
        _mm_body,
        grid=(N // blk,),
        in_specs=[
            pl.BlockSpec((blk, D), lambda i: (i, 0)),
            pl.BlockSpec((D, D), lambda i: (0, 0)),
            pl.BlockSpec((2, blk, 16), lambda i: (0, i, 0)),
        ],
        out_specs=[
            pl.BlockSpec((2, blk, HALF), lambda i: (0, i, 0)),
            pl.BlockSpec((blk, D), lambda i: (i, 0)),
        ],
        out_shape=[
            jax.ShapeDtypeStruct((2, N, HALF), jnp.float32),
            jax.ShapeDtypeStruct((N, D), jnp.float32),
        ],
    )(x, W, degp)


def _bn_body(agg_ref, degp_ref, self_ref, b_ref, g_ref, be_ref, o_ref):
    deg = degp_ref[0, :N, 0:1] + degp_ref[1, :N, 0:1] + 1.0
    dis = lax.rsqrt(deg)
    out = agg_ref[0, :N, :] * dis + self_ref[...] + b_ref[...]
    m = jnp.mean(out, axis=0, keepdims=True)
    v = jnp.mean((out - m) * (out - m), axis=0, keepdims=True)
    xn = (out - m) / jnp.sqrt(v + 1e-5)
    y = g_ref[...] * xn + be_ref[...]
    o_ref[...] = jnp.maximum(y, 0.0)


def _bn(agg, degp, self_t, b2, g2, be2):
    return pl.pallas_call(
        _bn_body,
        grid=(2,),
        in_specs=[
            pl.BlockSpec((1, ROWS, HALF), lambda i: (i, 0, 0)),
            pl.BlockSpec((2, ROWS, 16), lambda i: (0, 0, 0)),
            pl.BlockSpec((N, HALF), lambda i: (0, i)),
            pl.BlockSpec((1, HALF), lambda i: (0, i)),
            pl.BlockSpec((1, HALF), lambda i: (0, i)),
            pl.BlockSpec((1, HALF), lambda i: (0, i)),
        ],
        out_specs=pl.BlockSpec((N, HALF), lambda i: (0, i)),
        out_shape=jax.ShapeDtypeStruct((N, D), jnp.float32),
    )(agg, degp, self_t, b2, g2, be2)


def kernel(x, edge_index, W, b, gamma, beta):
    src = edge_index[0].astype(jnp.int32)
    dst = edge_index[1].astype(jnp.int32)
    pad = E_PAD - E
    src_p = jnp.concatenate([src, jnp.zeros((pad,), jnp.int32)])
    dst_p = jnp.concatenate([dst, jnp.full((pad,), PAD_DST, jnp.int32)])
    src2 = jnp.stack([src_p, src_p + N]).reshape(2 * 1280, CHUNK)
    dst2 = dst_p.reshape(1280, CHUNK)

    degp = _deg(dst2).reshape(2, ROWS, 16)
    hs, self_t = _mm(x, W, degp)
    agg = _agg(src2, dst2, hs.reshape(2 * N, HALF)).reshape(2, ROWS, HALF)
    return _bn(
        agg, degp, self_t,
        b.reshape(1, D), gamma.reshape(1, D), beta.reshape(1, D),
    )
